# baseline (speedup 1.0000x reference)
import functools

import jax
import jax.numpy as jnp
from jax import lax
from jax.experimental import pallas as pl
from jax.experimental.pallas import tpu as pltpu
from jax.experimental.pallas import tpu_sc as plsc

_D = 64
_B = 4096 * 200
_NC = 2
_NS = 16
_NW = _NC * _NS
_CH = 128
_ROWS_PER_W = _B // _NW
_NCH = _ROWS_PER_W // _CH


def _make_emb():
    mesh = plsc.VectorSubcoreMesh(core_axis_name="c", subcore_axis_name="s")

    @functools.partial(
        pl.kernel,
        mesh=mesh,
        out_type=jax.ShapeDtypeStruct((_B, _D), jnp.float32),
        scratch_types=[
            pltpu.VMEM((_NCH, _CH), jnp.int32),
            pltpu.VMEM((_CH, _D), jnp.float32),
            pltpu.SemaphoreType.DMA,
        ],
        compiler_params=pltpu.CompilerParams(use_tc_tiling_on_sc=False),
    )
    def emb(idx_hbm, table_hbm, out_hbm, idx_v, rows_v, sem):
        wid = lax.axis_index("s") * _NC + lax.axis_index("c")
        pltpu.sync_copy(idx_hbm.at[pl.ds(wid * _NCH, _NCH)], idx_v)
        base = wid * _ROWS_PER_W

        def body(j, carry):
            pltpu.async_copy(table_hbm.at[idx_v.at[j]], rows_v, sem).wait()
            pltpu.sync_copy(rows_v, out_hbm.at[pl.ds(base + j * _CH, _CH)])
            return carry

        lax.fori_loop(0, _NCH, body, 0)

    return emb


_emb = _make_emb()


def kernel(x, table):
    idx = x.reshape(_B // _CH, _CH)
    out = _emb(idx, table)
    return out.reshape(x.shape[0], x.shape[1], _D)


# speedup vs baseline: 1.1164x; 1.1164x over previous
import functools

import jax
import jax.numpy as jnp
from jax import lax
from jax.experimental import pallas as pl
from jax.experimental.pallas import tpu as pltpu
from jax.experimental.pallas import tpu_sc as plsc

_D = 64
_B = 4096 * 200
_NC = 2
_NS = 16
_NW = _NC * _NS
_CH = 128
_ROWS_PER_W = _B // _NW
_NCH = _ROWS_PER_W // _CH
_K = 4
_NG = _NCH // _K


def _make_emb():
    mesh = plsc.VectorSubcoreMesh(core_axis_name="c", subcore_axis_name="s")

    @functools.partial(
        pl.kernel,
        mesh=mesh,
        out_type=jax.ShapeDtypeStruct((_B, _D), jnp.float32),
        scratch_types=[
            pltpu.VMEM((_NCH, _CH), jnp.int32),
            pltpu.VMEM((2, _K, _CH, _D), jnp.float32),
            pltpu.SemaphoreType.DMA,
            pltpu.SemaphoreType.DMA,
        ],
        compiler_params=pltpu.CompilerParams(use_tc_tiling_on_sc=False),
    )
    def emb(idx_hbm, table_hbm, out_hbm, idx_v, rows_v, gsem, osem):
        wid = lax.axis_index("s") * _NC + lax.axis_index("c")
        pltpu.sync_copy(idx_hbm.at[pl.ds(wid * _NCH, _NCH)], idx_v)
        base = wid * _ROWS_PER_W

        def fire_g(s, g):
            for b in range(_K):
                pltpu.async_copy(table_hbm.at[idx_v.at[g * _K + b]],
                                 rows_v.at[s, b], gsem)

        def drain_g(s):
            for b in range(_K):
                pltpu.make_async_copy(table_hbm.at[idx_v.at[0]],
                                      rows_v.at[s, b], gsem).wait()

        def fire_o(s, g):
            for b in range(_K):
                pltpu.async_copy(rows_v.at[s, b],
                                 out_hbm.at[pl.ds(base + (g * _K + b) * _CH, _CH)],
                                 osem)

        def drain_o(s):
            for b in range(_K):
                pltpu.make_async_copy(rows_v.at[s, b],
                                      out_hbm.at[pl.ds(base, _CH)], osem).wait()

        fire_g(0, 0)
        fire_g(1, 1)
        drain_g(0)
        fire_o(0, 0)
        drain_g(1)
        fire_o(1, 1)

        def body(p, carry):
            for s in range(2):
                g = 2 * p + s
                drain_o(s)
                fire_g(s, g)
                drain_g(s)
                fire_o(s, g)
            return carry

        lax.fori_loop(1, _NG // 2, body, 0)
        drain_o(0)
        drain_o(1)

    return emb


_emb = _make_emb()


def kernel(x, table):
    idx = x.reshape(_B // _CH, _CH)
    out = _emb(idx, table)
    return out.reshape(x.shape[0], x.shape[1], _D)
